# Initial kernel scaffold; baseline (speedup 1.0000x reference)
import jax
import jax.numpy as jnp
from jax.experimental import pallas as pl


def kernel(route_codes_btr, table_ngram_2, table_ngram_3):
    raise NotImplementedError("write your pallas kernel here")


# speedup vs baseline: 2.2707x; 2.2707x over previous
import jax
import jax.numpy as jnp
from jax import lax
from jax.experimental import pallas as pl
from jax.experimental.pallas import tpu as pltpu
from jax.experimental.pallas import tpu_sc as plsc

S = 2048
R = 128
A = 16
MEM = 64
NC, NS = 2, 16
NW = NC * NS
P_PER_W = S // NW
K_PER_W = P_PER_W * R
HIST = 2 * R
CHUNK = K_PER_W + HIST


def _sc_body(codes_hbm, t2_hbm, t3_hbm, out_hbm,
             codes_v, idx2_v, idx3_v, buf_v, zbuf_v, sem):
    cid = lax.axis_index("c")
    sid = lax.axis_index("s")
    wid = sid * NC + cid
    base_k = wid * K_PER_W
    start = pl.multiple_of(jnp.maximum(base_k - HIST, 0), HIST)
    off = base_k - start

    pltpu.sync_copy(codes_hbm.at[pl.ds(start, CHUNK)], codes_v)

    lanes = lax.iota(jnp.int32, 16)

    @pl.loop(0, P_PER_W)
    def _(p):
        for jj in range(R // 16):
            loc = off + p * R + jj * 16
            c0 = codes_v[pl.ds(loc, 16)]
            c1 = codes_v[pl.ds(jnp.maximum(loc - R, 0), 16)]
            c2 = codes_v[pl.ds(jnp.maximum(loc - 2 * R, 0), 16)]
            rvec = lanes + (jj * 16)
            idx2_v[p, pl.ds(jj * 16, 16)] = rvec * (A * A) + c1 + c0 * A
            idx3_v[p, pl.ds(jj * 16, 16)] = (
                rvec * (A * A * A) + c2 + c1 * A + c0 * (A * A))

    @pl.when(wid == 0)
    def _():
        @pl.loop(0, 2 * R)
        def _(i):
            for c in range(MEM // 16):
                zbuf_v[i, pl.ds(c * 16, 16)] = jnp.zeros((16,), jnp.float32)
        pltpu.sync_copy(zbuf_v, out_hbm.at[pl.ds(0, 2 * R)])
        pltpu.async_copy(t2_hbm.at[idx2_v.at[1]],
                         zbuf_v.at[pl.ds(0, R)], sem).wait()
        pltpu.sync_copy(zbuf_v, out_hbm.at[pl.ds(2 * R, 2 * R)])

    p_lo = jnp.where(wid == 0, 2, 0)

    @pl.loop(p_lo, P_PER_W)
    def _(p):
        s = wid * P_PER_W + p
        cp2 = pltpu.async_copy(t2_hbm.at[idx2_v.at[p]],
                               buf_v.at[pl.ds(0, R)], sem)
        cp3 = pltpu.async_copy(t3_hbm.at[idx3_v.at[p]],
                               buf_v.at[pl.ds(R, R)], sem)
        cp2.wait()
        cp3.wait()
        pltpu.sync_copy(buf_v, out_hbm.at[pl.ds(s * 2 * R, 2 * R)])


_mesh = plsc.VectorSubcoreMesh(core_axis_name="c", subcore_axis_name="s",
                               num_cores=NC, num_subcores=NS)

_sc_call = pl.kernel(
    _sc_body,
    out_type=jax.ShapeDtypeStruct((S * 2 * R, MEM), jnp.float32),
    mesh=_mesh,
    scratch_types=[
        pltpu.VMEM((CHUNK,), jnp.int32),
        pltpu.VMEM((P_PER_W, R), jnp.int32),
        pltpu.VMEM((P_PER_W, R), jnp.int32),
        pltpu.VMEM((2 * R, MEM), jnp.float32),
        pltpu.VMEM((2 * R, MEM), jnp.float32),
        pltpu.SemaphoreType.DMA,
    ],
    compiler_params=pltpu.CompilerParams(use_tc_tiling_on_sc=False),
)


@jax.jit
def _run(codes_flat, t2, t3):
    return _sc_call(codes_flat, t2, t3)


def kernel(route_codes_btr, table_ngram_2, table_ngram_3):
    b, s, r = route_codes_btr.shape
    assert (b, s, r) == (1, S, R)
    codes_flat = route_codes_btr.astype(jnp.int32).reshape(-1)
    out = _run(codes_flat, table_ngram_2, table_ngram_3)
    return out.reshape(1, S, 2 * R * MEM)


# speedup vs baseline: 2.4465x; 1.0774x over previous
import jax
import jax.numpy as jnp
from jax import lax
from jax.experimental import pallas as pl
from jax.experimental.pallas import tpu as pltpu
from jax.experimental.pallas import tpu_sc as plsc

S = 2048
R = 128
A = 16
MEM = 64
NC, NS = 2, 16
NW = NC * NS
P_PER_W = S // NW
K_PER_W = P_PER_W * R
HIST = 2 * R
CHUNK = K_PER_W + HIST


def _sc_body(codes_hbm, t2_hbm, t3_hbm, out_hbm,
             codes_v, idx2_v, idx3_v, buf_v, zbuf_v, gsem, ssem):
    cid = lax.axis_index("c")
    sid = lax.axis_index("s")
    wid = sid * NC + cid
    base_k = wid * K_PER_W
    start = pl.multiple_of(jnp.maximum(base_k - HIST, 0), HIST)
    off = base_k - start
    base_row = wid * P_PER_W * 2 * R

    pltpu.sync_copy(codes_hbm.at[pl.ds(start, CHUNK)], codes_v)

    lanes = lax.iota(jnp.int32, 16)

    @pl.loop(0, P_PER_W)
    def _(p):
        for jj in range(R // 16):
            loc = off + p * R + jj * 16
            c0 = codes_v[pl.ds(loc, 16)]
            c1 = codes_v[pl.ds(jnp.maximum(loc - R, 0), 16)]
            c2 = codes_v[pl.ds(jnp.maximum(loc - 2 * R, 0), 16)]
            rvec = lanes + (jj * 16)
            idx2_v[p, pl.ds(jj * 16, 16)] = rvec * (A * A) + c1 + c0 * A
            idx3_v[p, pl.ds(jj * 16, 16)] = (
                rvec * (A * A * A) + c2 + c1 * A + c0 * (A * A))

    @pl.when(wid == 0)
    def _():
        @pl.loop(0, R)
        def _(i):
            for c in range(MEM // 16):
                zbuf_v[i, pl.ds(c * 16, 16)] = jnp.zeros((16,), jnp.float32)

    def issue_gathers(p, b):
        pltpu.async_copy(t2_hbm.at[idx2_v.at[p]],
                         buf_v.at[b, pl.ds(0, R)], gsem)
        pltpu.async_copy(t3_hbm.at[idx3_v.at[p]],
                         buf_v.at[b, pl.ds(R, R)], gsem)

    def drain(sem, b):
        pltpu.make_async_copy(out_hbm.at[pl.ds(0, 2 * R)],
                              buf_v.at[b], sem).wait()

    issue_gathers(0, 0)
    issue_gathers(1, 1)

    @pl.loop(0, P_PER_W, step=2)
    def _(p):
        for b in range(2):
            q = p + b
            drain(gsem, b)
            pltpu.async_copy(buf_v.at[b],
                             out_hbm.at[pl.ds(base_row + q * 2 * R, 2 * R)],
                             ssem)
            drain(ssem, b)

            @pl.when(q + 2 < P_PER_W)
            def _():
                issue_gathers(q + 2, b)

    @pl.when(wid == 0)
    def _():
        pltpu.sync_copy(zbuf_v, out_hbm.at[pl.ds(0, R)])
        pltpu.sync_copy(zbuf_v, out_hbm.at[pl.ds(R, R)])
        pltpu.sync_copy(zbuf_v, out_hbm.at[pl.ds(2 * R + R, R)])


_mesh = plsc.VectorSubcoreMesh(core_axis_name="c", subcore_axis_name="s",
                               num_cores=NC, num_subcores=NS)

_sc_call = pl.kernel(
    _sc_body,
    out_type=jax.ShapeDtypeStruct((S * 2 * R, MEM), jnp.float32),
    mesh=_mesh,
    scratch_types=[
        pltpu.VMEM((CHUNK,), jnp.int32),
        pltpu.VMEM((P_PER_W, R), jnp.int32),
        pltpu.VMEM((P_PER_W, R), jnp.int32),
        pltpu.VMEM((2, 2 * R, MEM), jnp.float32),
        pltpu.VMEM((R, MEM), jnp.float32),
        pltpu.SemaphoreType.DMA,
        pltpu.SemaphoreType.DMA,
    ],
    compiler_params=pltpu.CompilerParams(use_tc_tiling_on_sc=False),
)


@jax.jit
def _run(codes_flat, t2, t3):
    return _sc_call(codes_flat, t2, t3)


def kernel(route_codes_btr, table_ngram_2, table_ngram_3):
    b, s, r = route_codes_btr.shape
    assert (b, s, r) == (1, S, R)
    codes_flat = route_codes_btr.astype(jnp.int32).reshape(-1)
    out = _run(codes_flat, table_ngram_2, table_ngram_3)
    return out.reshape(1, S, 2 * R * MEM)


# speedup vs baseline: 3.2377x; 1.3234x over previous
import jax
import jax.numpy as jnp
from jax import lax
from jax.experimental import pallas as pl
from jax.experimental.pallas import tpu as pltpu
from jax.experimental.pallas import tpu_sc as plsc

S = 2048
R = 128
A = 16
MEM = 64
NC, NS = 2, 16
NW = NC * NS
P_PER_W = S // NW
K_PER_W = P_PER_W * R
HIST = 2 * R
CHUNK = K_PER_W + HIST
TR = S // 8
RH = R // 2


def _sc_body(codes_hbm, t2_hbm, t3_hbm, out_hbm,
             codes_v, idx2_v, idx3_v, gbuf_v, zbuf_v, gsem, ssem):
    cid = lax.axis_index("c")
    sid = lax.axis_index("s")
    wid = sid * NC + cid
    base_k = wid * K_PER_W
    start = pl.multiple_of(jnp.maximum(base_k - HIST, 0), HIST)
    off = base_k - start

    pltpu.sync_copy(codes_hbm.at[pl.ds(start, CHUNK)], codes_v)

    lanes2 = lax.iota(jnp.int32, 16) * 2

    @pl.loop(0, P_PER_W)
    def _(p):
        for rb in range(2):
            for jjr in range(RH // 16):
                kvec = (off + p * R + rb + jjr * 32) + lanes2
                c0 = plsc.load_gather(codes_v, [kvec])
                c1 = plsc.load_gather(codes_v, [jnp.maximum(kvec - R, 0)])
                c2 = plsc.load_gather(codes_v, [jnp.maximum(kvec - 2 * R, 0)])
                rvec = (lanes2 + (jjr * 32 + rb))
                idx2_v[p, rb, pl.ds(jjr * 16, 16)] = (
                    rvec * (A * A) + c1 + c0 * A)
                idx3_v[p, rb, pl.ds(jjr * 16, 16)] = (
                    rvec * (A * A * A) + c2 + c1 * A + c0 * (A * A))

    @pl.when(wid == 0)
    def _():
        @pl.loop(0, MEM)
        def _(i):
            for c in range(2 * MEM // 16):
                zbuf_v[i, pl.ds(c * 16, 16)] = jnp.zeros((16,), jnp.float32)

    def issue_gathers(p, b):
        pltpu.async_copy(t2_hbm.at[idx2_v.at[p, 0]], gbuf_v.at[b, 0, 0], gsem)
        pltpu.async_copy(t2_hbm.at[idx2_v.at[p, 1]], gbuf_v.at[b, 0, 1], gsem)
        pltpu.async_copy(t3_hbm.at[idx3_v.at[p, 0]], gbuf_v.at[b, 1, 0], gsem)
        pltpu.async_copy(t3_hbm.at[idx3_v.at[p, 1]], gbuf_v.at[b, 1, 1], gsem)

    def drain(sem, tile_r, srem, b):
        pltpu.make_async_copy(out_hbm.at[tile_r, :, :, srem, pl.ds(0, MEM)],
                              gbuf_v.at[b, 0], sem).wait()
        pltpu.make_async_copy(out_hbm.at[tile_r, :, :, srem, pl.ds(0, MEM)],
                              gbuf_v.at[b, 1], sem).wait()

    issue_gathers(0, 0)
    issue_gathers(1, 1)

    @pl.loop(0, P_PER_W, step=2)
    def _(p):
        for b in range(2):
            q = p + b
            tile_r = wid * (P_PER_W // 8) + q // 8
            srem = q % 8
            drain(gsem, tile_r, srem, b)
            for o in range(2):
                for rb in range(2):
                    pltpu.async_copy(
                        gbuf_v.at[b, o, rb],
                        out_hbm.at[tile_r, o, :, srem, pl.ds(rb * MEM, MEM)],
                        ssem)
            drain(ssem, tile_r, srem, b)

            @pl.when(q + 2 < P_PER_W)
            def _():
                issue_gathers(q + 2, b)

    @pl.when(wid == 0)
    def _():
        pltpu.sync_copy(zbuf_v, out_hbm.at[0, 0, :, 0, :])
        pltpu.sync_copy(zbuf_v, out_hbm.at[0, 1, :, 0, :])
        pltpu.sync_copy(zbuf_v, out_hbm.at[0, 1, :, 1, :])


_mesh = plsc.VectorSubcoreMesh(core_axis_name="c", subcore_axis_name="s",
                               num_cores=NC, num_subcores=NS)

_sc_call = pl.kernel(
    _sc_body,
    out_type=jax.ShapeDtypeStruct((TR, 2, MEM, 8, 2 * MEM), jnp.float32),
    mesh=_mesh,
    scratch_types=[
        pltpu.VMEM((CHUNK,), jnp.int32),
        pltpu.VMEM((P_PER_W, 2, RH), jnp.int32),
        pltpu.VMEM((P_PER_W, 2, RH), jnp.int32),
        pltpu.VMEM((2, 2, 2, RH, MEM), jnp.float32),
        pltpu.VMEM((MEM, 2 * MEM), jnp.float32),
        pltpu.SemaphoreType.DMA,
        pltpu.SemaphoreType.DMA,
    ],
    compiler_params=pltpu.CompilerParams(use_tc_tiling_on_sc=False,
                                         needs_layout_passes=False),
)


@jax.jit
def _run(codes_flat, t2, t3):
    return _sc_call(codes_flat, t2, t3)


def kernel(route_codes_btr, table_ngram_2, table_ngram_3):
    b, s, r = route_codes_btr.shape
    assert (b, s, r) == (1, S, R)
    codes_flat = route_codes_btr.astype(jnp.int32).reshape(-1)
    out5 = _run(codes_flat, table_ngram_2, table_ngram_3)
    return out5.transpose(0, 3, 1, 2, 4).reshape(1, S, 2 * R * MEM)
